# baseline (speedup 1.0000x reference)
import functools

import numpy as np
import jax
import jax.numpy as jnp
from jax import lax
from jax.experimental import pallas as pl
from jax.experimental.pallas import tpu as pltpu
from jax.experimental.pallas import tpu_sc as plsc

VOCAB = 100000
EMB = 128
B = 4096
L = 200

NC = 2
NS = 16
NW = NC * NS
BPW = B // NW
LANES = 16
NPACK = EMB // 2
NWORDV = NPACK // LANES
SPLITS = ((0, 104), (104, 96))

_PERM = np.empty((EMB,), np.int32)
for _j in range(EMB):
    _k, _q0, _r = _j // 32, (_j // 16) % 2, _j % 16
    _PERM[_j] = 32 * _k + 2 * _r + _q0


def _pool_body(src_hbm, table_hbm, out_hbm, idx_all, rows0, rows1,
               out_stage, sem0, sem1):
    wid = lax.axis_index("s") * NC + lax.axis_index("c")
    base = wid * BPW

    pltpu.sync_copy(src_hbm.at[pl.ds(base * L, BPW * L)], idx_all)

    rows = (rows0, rows1)
    sems = (sem0, sem1)

    def issue(i, buf):
        for off, n in SPLITS:
            pltpu.async_copy(
                table_hbm.at[idx_all.at[pl.ds(i * L + off, n)]],
                rows[buf].at[pl.ds(off, n)],
                sems[buf],
            )

    def drain(i, buf):
        for off, n in SPLITS:
            pltpu.make_async_copy(
                table_hbm.at[idx_all.at[pl.ds(i * L + off, n)]],
                rows[buf].at[pl.ds(off, n)],
                sems[buf],
            ).wait()

    issue(0, 0)
    issue(1, 1)

    def step(i0, carry):
        for buf in range(2):
            i = i0 * 2 + buf
            drain(i, buf)

            def body(r, acc):
                row = rows[buf].at[r]
                new = []
                for k in range(NWORDV):
                    w = row[pl.ds(k * LANES, LANES)]
                    even = lax.bitcast_convert_type(w << 16, jnp.float32)
                    odd = lax.bitcast_convert_type(
                        w & jnp.int32(-65536), jnp.float32)
                    new.append(acc[2 * k] + even)
                    new.append(acc[2 * k + 1] + odd)
                return tuple(new)

            zeros = tuple(
                jnp.zeros((LANES,), jnp.float32) for _ in range(2 * NWORDV))
            acc = lax.fori_loop(0, L, body, zeros, unroll=2)
            for q in range(2 * NWORDV):
                out_stage[i, pl.ds(q * LANES, LANES)] = acc[q]

            @pl.when(i + 2 < BPW)
            def _():
                issue(i + 2, buf)
        return carry

    lax.fori_loop(0, BPW // 2, step, 0)

    pltpu.sync_copy(out_stage, out_hbm.at[pl.ds(base, BPW)])


def _sc_pool(src32, table_packed):
    mesh = plsc.VectorSubcoreMesh(core_axis_name="c", subcore_axis_name="s")
    f = pl.kernel(
        _pool_body,
        out_type=jax.ShapeDtypeStruct((B, EMB), jnp.float32),
        mesh=mesh,
        scratch_types=[
            pltpu.VMEM((BPW * L,), jnp.int32),
            pltpu.VMEM((L, NPACK), jnp.int32),
            pltpu.VMEM((L, NPACK), jnp.int32),
            pltpu.VMEM((BPW, EMB), jnp.float32),
            pltpu.SemaphoreType.DMA,
            pltpu.SemaphoreType.DMA,
        ],
        compiler_params=pltpu.CompilerParams(use_tc_tiling_on_sc=False),
    )
    return f(src32, table_packed)


def _ffn_body(x_ref, w_ref, b_ref, o_ref):
    x = x_ref[...] * (1.0 / L)
    o_ref[...] = lax.dot_general(
        x, w_ref[...], (((1,), (1,)), ((), ())),
        preferred_element_type=jnp.float32) + b_ref[...]


def _tc_ffn(sums, Wp, b):
    blk = 512
    grid = (B // blk,)
    return pl.pallas_call(
        _ffn_body,
        grid=grid,
        in_specs=[
            pl.BlockSpec((blk, EMB), lambda i: (i, 0)),
            pl.BlockSpec((EMB, EMB), lambda i: (0, 0)),
            pl.BlockSpec((1, EMB), lambda i: (0, 0)),
        ],
        out_specs=pl.BlockSpec((blk, EMB), lambda i: (i, 0)),
        out_shape=jax.ShapeDtypeStruct((B, EMB), jnp.float32),
    )(sums, Wp, b.reshape(1, EMB))


def _pack_table(table):
    u = lax.bitcast_convert_type(
        table.astype(jnp.bfloat16), jnp.uint16).astype(jnp.uint32)
    packed = u[:, 0::2] | (u[:, 1::2] << 16)
    return lax.bitcast_convert_type(packed, jnp.int32)


@jax.jit
def kernel(src, table, W, b):
    src32 = src.astype(jnp.int32).reshape(B * L)
    table_packed = _pack_table(table)
    sums = _sc_pool(src32, table_packed)
    hidden = _tc_ffn(sums, W[:, _PERM], b)
    return hidden[None, :, :]


# speedup vs baseline: 3.1931x; 3.1931x over previous
import functools

import numpy as np
import jax
import jax.numpy as jnp
from jax import lax
from jax.experimental import pallas as pl
from jax.experimental.pallas import tpu as pltpu
from jax.experimental.pallas import tpu_sc as plsc

VOCAB = 100000
EMB = 128
B = 4096
L = 200

NC = 2
NS = 16
NW = NC * NS
BPW = B // NW
LANES = 16
NPACK = EMB // 2
NWORDV = NPACK // LANES
SPLITS = ((0, 104), (104, 96))

_PERM = np.empty((EMB,), np.int32)
for _j in range(EMB):
    _k, _q0, _r = _j // 32, (_j // 16) % 2, _j % 16
    _PERM[_j] = 32 * _k + 2 * _r + _q0


def _pool_body(src_hbm, table_hbm, out_hbm, idx_all, rows0, rows1,
               out_stage, sem0, sem1):
    wid = lax.axis_index("s") * NC + lax.axis_index("c")
    base = wid * BPW

    pltpu.sync_copy(src_hbm.at[pl.ds(base * L, BPW * L)], idx_all)

    rows = (rows0, rows1)
    sems = (sem0, sem1)

    def issue(i, buf):
        for off, n in SPLITS:
            pltpu.async_copy(
                table_hbm.at[idx_all.at[pl.ds(i * L + off, n)]],
                rows[buf].at[pl.ds(off, n)],
                sems[buf],
            )

    def drain(i, buf):
        for off, n in SPLITS:
            pltpu.make_async_copy(
                table_hbm.at[idx_all.at[pl.ds(i * L + off, n)]],
                rows[buf].at[pl.ds(off, n)],
                sems[buf],
            ).wait()

    issue(0, 0)
    issue(1, 1)

    def step(i0, carry):
        for buf in range(2):
            i = i0 * 2 + buf
            drain(i, buf)

            def body(r, acc):
                row = rows[buf].at[r]
                new = []
                for k in range(NWORDV):
                    w = row[pl.ds(k * LANES, LANES)]
                    even = lax.bitcast_convert_type(w << 16, jnp.float32)
                    odd = lax.bitcast_convert_type(
                        w & jnp.int32(-65536), jnp.float32)
                    new.append(acc[2 * k] + even)
                    new.append(acc[2 * k + 1] + odd)
                return tuple(new)

            zeros = tuple(
                jnp.zeros((LANES,), jnp.float32) for _ in range(2 * NWORDV))
            acc = lax.fori_loop(0, L, body, zeros, unroll=2)
            for q in range(2 * NWORDV):
                out_stage[i, pl.ds(q * LANES, LANES)] = acc[q]

            @pl.when(i + 2 < BPW)
            def _():
                issue(i + 2, buf)
        return carry

    lax.fori_loop(0, BPW // 2, step, 0)

    pltpu.sync_copy(out_stage, out_hbm.at[pl.ds(base, BPW)])


def _sc_pool(src32, table_packed):
    mesh = plsc.VectorSubcoreMesh(core_axis_name="c", subcore_axis_name="s")
    f = pl.kernel(
        _pool_body,
        out_type=jax.ShapeDtypeStruct((B, EMB), jnp.float32),
        mesh=mesh,
        scratch_types=[
            pltpu.VMEM((BPW * L,), jnp.int32),
            pltpu.VMEM((L, NPACK), jnp.int32),
            pltpu.VMEM((L, NPACK), jnp.int32),
            pltpu.VMEM((BPW, EMB), jnp.float32),
            pltpu.SemaphoreType.DMA,
            pltpu.SemaphoreType.DMA,
        ],
        compiler_params=pltpu.CompilerParams(use_tc_tiling_on_sc=False),
    )
    return f(src32, table_packed)


def _ffn_body(x_ref, w_ref, b_ref, o_ref):
    x = x_ref[...] * (1.0 / L)
    o_ref[...] = lax.dot_general(
        x, w_ref[...], (((1,), (1,)), ((), ())),
        preferred_element_type=jnp.float32) + b_ref[...]


def _tc_ffn(sums, Wp, b):
    blk = 512
    grid = (B // blk,)
    return pl.pallas_call(
        _ffn_body,
        grid=grid,
        in_specs=[
            pl.BlockSpec((blk, EMB), lambda i: (i, 0)),
            pl.BlockSpec((EMB, EMB), lambda i: (0, 0)),
            pl.BlockSpec((1, EMB), lambda i: (0, 0)),
        ],
        out_specs=pl.BlockSpec((blk, EMB), lambda i: (i, 0)),
        out_shape=jax.ShapeDtypeStruct((B, EMB), jnp.float32),
    )(sums, Wp, b.reshape(1, EMB))


def _pack_table(table):
    tb = table.astype(jnp.bfloat16).reshape(VOCAB, NPACK, 2)
    return lax.bitcast_convert_type(tb, jnp.int32)


@jax.jit
def kernel(src, table, W, b):
    src32 = src.astype(jnp.int32).reshape(B * L)
    table_packed = _pack_table(table)
    sums = _sc_pool(src32, table_packed)
    hidden = _tc_ffn(sums, W[:, _PERM], b)
    return hidden[None, :, :]


# speedup vs baseline: 8.1934x; 2.5660x over previous
import functools

import numpy as np
import jax
import jax.numpy as jnp
from jax import lax
from jax.experimental import pallas as pl
from jax.experimental.pallas import tpu as pltpu
from jax.experimental.pallas import tpu_sc as plsc

VOCAB = 100000
EMB = 128
B = 4096
L = 200

NC = 2
NS = 16
NW = NC * NS
BPW = B // NW
LANES = 16
NPACK = EMB // 2
NWORDV = NPACK // LANES
SPLITS = ((0, 104), (104, 96))


def _pool_body(src_hbm, table_hbm, out_hbm, idx_all, rows0, rows1,
               out_stage, sem0, sem1):
    wid = lax.axis_index("s") * NC + lax.axis_index("c")
    base = wid * BPW

    pltpu.sync_copy(src_hbm.at[pl.ds(base * L, BPW * L)], idx_all)

    rows = (rows0, rows1)
    sems = (sem0, sem1)

    def issue(i, buf):
        for off, n in SPLITS:
            pltpu.async_copy(
                table_hbm.at[idx_all.at[pl.ds(i * L + off, n)]],
                rows[buf].at[pl.ds(off, n)],
                sems[buf],
            )

    def drain(i, buf):
        for off, n in SPLITS:
            pltpu.make_async_copy(
                table_hbm.at[idx_all.at[pl.ds(i * L + off, n)]],
                rows[buf].at[pl.ds(off, n)],
                sems[buf],
            ).wait()

    issue(0, 0)
    issue(1, 1)

    def step(i0, carry):
        for buf in range(2):
            i = i0 * 2 + buf
            drain(i, buf)

            def body(r, acc):
                row = rows[buf].at[r]
                new = []
                for k in range(NWORDV):
                    w = row[pl.ds(k * LANES, LANES)]
                    lo = lax.bitcast_convert_type(w << 16, jnp.float32)
                    hi = lax.bitcast_convert_type(
                        w & jnp.int32(-65536), jnp.float32)
                    new.append(acc[2 * k] + lo)
                    new.append(acc[2 * k + 1] + hi)
                return tuple(new)

            zeros = tuple(
                jnp.zeros((LANES,), jnp.float32) for _ in range(2 * NWORDV))
            acc = lax.fori_loop(0, L, body, zeros, unroll=2)
            for q in range(2 * NWORDV):
                out_stage[i, pl.ds(q * LANES, LANES)] = acc[q]

            @pl.when(i + 2 < BPW)
            def _():
                issue(i + 2, buf)
        return carry

    lax.fori_loop(0, BPW // 2, step, 0)

    pltpu.sync_copy(out_stage, out_hbm.at[pl.ds(base, BPW)])


def _sc_pool(src32, table_packed):
    mesh = plsc.VectorSubcoreMesh(core_axis_name="c", subcore_axis_name="s")
    f = pl.kernel(
        _pool_body,
        out_type=jax.ShapeDtypeStruct((B, EMB), jnp.float32),
        mesh=mesh,
        scratch_types=[
            pltpu.VMEM((BPW * L,), jnp.int32),
            pltpu.VMEM((L, NPACK), jnp.int32),
            pltpu.VMEM((L, NPACK), jnp.int32),
            pltpu.VMEM((BPW, EMB), jnp.float32),
            pltpu.SemaphoreType.DMA,
            pltpu.SemaphoreType.DMA,
        ],
        compiler_params=pltpu.CompilerParams(use_tc_tiling_on_sc=False),
    )
    return f(src32, table_packed)


def _ffn_body(x_ref, w_ref, b_ref, o_ref):
    x = x_ref[...] * (1.0 / L)
    o_ref[...] = lax.dot_general(
        x, w_ref[...], (((1,), (1,)), ((), ())),
        preferred_element_type=jnp.float32) + b_ref[...]


def _tc_ffn(sums, Wp, b):
    blk = 512
    grid = (B // blk,)
    return pl.pallas_call(
        _ffn_body,
        grid=grid,
        in_specs=[
            pl.BlockSpec((blk, EMB), lambda i: (i, 0)),
            pl.BlockSpec((EMB, EMB), lambda i: (0, 0)),
            pl.BlockSpec((1, EMB), lambda i: (0, 0)),
        ],
        out_specs=pl.BlockSpec((blk, EMB), lambda i: (i, 0)),
        out_shape=jax.ShapeDtypeStruct((B, EMB), jnp.float32),
    )(sums, Wp, b.reshape(1, EMB))


VPW = VOCAB // NW
CROWS = 125
NCCH = VPW // CROWS


def _f32_to_bf16_bits(u):
    return (u + 0x7FFF + ((u >> 16) & 1)) >> 16


def _pack_body(table_hbm, out_hbm, in0, in1, st0, st1, sem0, sem1):
    wid = lax.axis_index("s") * NC + lax.axis_index("c")
    base = wid * VPW

    ins = (in0, in1)
    sts = (st0, st1)
    sems = (sem0, sem1)

    def issue(c, buf):
        pltpu.async_copy(
            table_hbm.at[pl.ds(base + c * CROWS, CROWS)], ins[buf],
            sems[buf])

    def drain(c, buf):
        pltpu.make_async_copy(
            table_hbm.at[pl.ds(base + c * CROWS, CROWS)], ins[buf],
            sems[buf]).wait()

    issue(0, 0)
    issue(1, 1)

    def step(c0, carry):
        for buf in range(2):
            c = c0 * 2 + buf
            drain(c, buf)

            def body(r, carry2):
                row = ins[buf].at[r]
                for k in range(NWORDV):
                    a = row[pl.ds((2 * k) * LANES, LANES)]
                    bb = row[pl.ds((2 * k + 1) * LANES, LANES)]
                    ua = _f32_to_bf16_bits(
                        lax.bitcast_convert_type(a, jnp.int32))
                    ub = _f32_to_bf16_bits(
                        lax.bitcast_convert_type(bb, jnp.int32))
                    w = (ua & jnp.int32(0xFFFF)) | (ub << 16)
                    sts[buf][r, pl.ds(k * LANES, LANES)] = w
                return carry2

            lax.fori_loop(0, CROWS, body, 0, unroll=2)

            @pl.when(c + 2 < NCCH)
            def _():
                issue(c + 2, buf)
            pltpu.sync_copy(
                sts[buf], out_hbm.at[pl.ds(base + c * CROWS, CROWS)])
        return carry

    lax.fori_loop(0, NCCH // 2, step, 0)


def _sc_pack(table):
    mesh = plsc.VectorSubcoreMesh(core_axis_name="c", subcore_axis_name="s")
    f = pl.kernel(
        _pack_body,
        out_type=jax.ShapeDtypeStruct((VOCAB, NPACK), jnp.int32),
        mesh=mesh,
        scratch_types=[
            pltpu.VMEM((CROWS, EMB), jnp.float32),
            pltpu.VMEM((CROWS, EMB), jnp.float32),
            pltpu.VMEM((CROWS, NPACK), jnp.int32),
            pltpu.VMEM((CROWS, NPACK), jnp.int32),
            pltpu.SemaphoreType.DMA,
            pltpu.SemaphoreType.DMA,
        ],
        compiler_params=pltpu.CompilerParams(use_tc_tiling_on_sc=False),
    )
    return f(table)


@jax.jit
def kernel(src, table, W, b):
    src32 = src.astype(jnp.int32).reshape(B * L)
    table_packed = _sc_pack(table)
    sums = _sc_pool(src32, table_packed)
    hidden = _tc_ffn(sums, W, b)
    return hidden[None, :, :]
